# baseline (speedup 1.0000x reference)
import functools

import jax
import jax.numpy as jnp
from jax import lax
from jax.experimental import pallas as pl
from jax.experimental.pallas import tpu as pltpu
from jax.experimental.pallas import tpu_sc as plsc

_B = 16
_Q = 4096
_L = 16
_NC = 2
_QH = _Q // _NC
_NITER = _QH // _L

_LOG1P_COEFS = (
    6.944574454176289e-05,
    0.9962619482337939,
    -0.4664424386275648,
    0.2186654836622196,
    -0.05545931374207994,
)


def _ce_body(pred_hbm, tgt_hbm, objn_hbm, out_hbm,
             row_v, tgt_v, objn_v, stage_v, allrows_v, shared):
    c = lax.axis_index("c")
    s = lax.axis_index("s")
    pltpu.sync_copy(pred_hbm.at[s, pl.ds(c * (2 * _QH), 2 * _QH)], row_v)
    pltpu.sync_copy(tgt_hbm.at[s, pl.ds(c * _QH, _QH)], tgt_v)
    pltpu.sync_copy(objn_hbm, objn_v)

    iota = lax.broadcasted_iota(jnp.int32, (_L,), 0)
    sful = jnp.full((_L,), s, jnp.int32)
    my_numb = plsc.load_gather(objn_v, [sful])
    inv = (1.0 / _B) / my_numb.astype(jnp.float32)
    tok0 = iota + c * _QH
    two_iota = iota * 2

    def step(i, acc):
        idx0 = two_iota + i * (2 * _L)
        g0 = plsc.load_gather(row_v, [idx0])
        g1 = plsc.load_gather(row_v, [idx0 + 1])
        t = tgt_v[pl.ds(i * _L, _L)]
        m = jnp.maximum(g0, g1)
        e = jnp.exp(jnp.minimum(g0, g1) - m)
        lp = jnp.full((_L,), _LOG1P_COEFS[-1], jnp.float32)
        for q in _LOG1P_COEFS[-2::-1]:
            lp = lp * e + q
        pt = jnp.where(t == 0, g0, g1)
        ce = (m - pt) + lp
        return acc + jnp.where(tok0 + i * _L < my_numb, ce, 0.0)

    acc = lax.fori_loop(0, _NITER, step, jnp.zeros((_L,), jnp.float32))
    stage_v[...] = acc * inv
    pltpu.sync_copy(stage_v, shared.at[pl.ds(s * _L, _L)])
    plsc.subcore_barrier()

    @pl.when(s == 0)
    def _():
        pltpu.sync_copy(shared, allrows_v)
        tot = jnp.zeros((_L,), jnp.float32)
        for ss in range(_B):
            tot = tot + allrows_v[pl.ds(ss * _L, _L)]
        stage_v[...] = jnp.full((_L,), jnp.sum(tot), jnp.float32)
        pltpu.sync_copy(stage_v, out_hbm.at[c])


_sc_celoss = functools.partial(
    pl.kernel,
    out_type=jax.ShapeDtypeStruct((_NC, _L), jnp.float32),
    mesh=plsc.VectorSubcoreMesh(
        core_axis_name="c", subcore_axis_name="s", num_cores=_NC),
    compiler_params=pltpu.CompilerParams(needs_layout_passes=False),
    scratch_types=[
        pltpu.VMEM((2 * _QH,), jnp.float32),
        pltpu.VMEM((_QH,), jnp.int32),
        pltpu.VMEM((_B,), jnp.int32),
        pltpu.VMEM((_L,), jnp.float32),
        pltpu.VMEM((_B * _L,), jnp.float32),
        pltpu.VMEM_SHARED((_B * _L,), jnp.float32),
    ],
)(_ce_body)


def kernel(pred, target, object_num):
    pred2 = pred.reshape(_B, 2 * _Q)
    out = _sc_celoss(pred2, target.astype(jnp.int32),
                     object_num.astype(jnp.int32))
    return out[0, 0] + out[1, 0]


# speedup vs baseline: 1.2047x; 1.2047x over previous
import functools

import jax
import jax.numpy as jnp
from jax import lax
from jax.experimental import pallas as pl
from jax.experimental.pallas import tpu as pltpu
from jax.experimental.pallas import tpu_sc as plsc

_B = 16
_Q = 4096
_L = 16
_NCHUNK = 2
_QC = _Q // _NCHUNK
_NITER = _QC // _L

_LOG1P_COEFS = (
    6.944574454176289e-05,
    0.9962619482337939,
    -0.4664424386275648,
    0.2186654836622196,
    -0.05545931374207994,
)


def _ce_body(pred_hbm, tgt_hbm, objn_hbm, out_hbm,
             row_v, tgt_v, objn_v, stage_v, allrows_v, shared,
             sem_o, sem_p0, sem_t0, sem_p1, sem_t1):
    s = lax.axis_index("s")
    cp_o = pltpu.async_copy(objn_hbm, objn_v, sem_o)
    cp_p0 = pltpu.async_copy(
        pred_hbm.at[s, pl.ds(0, 2 * _QC)], row_v.at[pl.ds(0, 2 * _QC)], sem_p0)
    cp_t0 = pltpu.async_copy(
        tgt_hbm.at[s, pl.ds(0, _QC)], tgt_v.at[pl.ds(0, _QC)], sem_t0)
    cp_p1 = pltpu.async_copy(
        pred_hbm.at[s, pl.ds(2 * _QC, 2 * _QC)],
        row_v.at[pl.ds(2 * _QC, 2 * _QC)], sem_p1)
    cp_t1 = pltpu.async_copy(
        tgt_hbm.at[s, pl.ds(_QC, _QC)], tgt_v.at[pl.ds(_QC, _QC)], sem_t1)

    cp_o.wait()
    iota = lax.broadcasted_iota(jnp.int32, (_L,), 0)
    sful = jnp.full((_L,), s, jnp.int32)
    my_numb = plsc.load_gather(objn_v, [sful])
    inv = (1.0 / _B) / my_numb.astype(jnp.float32)
    two_iota = iota * 2

    def step(i, acc):
        idx0 = two_iota + i * (2 * _L)
        g0 = plsc.load_gather(row_v, [idx0])
        g1 = plsc.load_gather(row_v, [idx0 + 1])
        t = tgt_v[pl.ds(i * _L, _L)]
        m = jnp.maximum(g0, g1)
        e = jnp.exp(jnp.minimum(g0, g1) - m)
        lp = jnp.full((_L,), _LOG1P_COEFS[-1], jnp.float32)
        for q in _LOG1P_COEFS[-2::-1]:
            lp = lp * e + q
        pt = jnp.where(t == 0, g0, g1)
        ce = (m - pt) + lp
        return acc + jnp.where(iota + i * _L < my_numb, ce, 0.0)

    cp_p0.wait()
    cp_t0.wait()
    acc = lax.fori_loop(0, _NITER, step, jnp.zeros((_L,), jnp.float32))
    cp_p1.wait()
    cp_t1.wait()
    acc = lax.fori_loop(_NITER, 2 * _NITER, step, acc)

    stage_v[...] = acc * inv
    pltpu.sync_copy(stage_v, shared.at[pl.ds(s * _L, _L)])
    plsc.subcore_barrier()

    @pl.when(s == 0)
    def _():
        pltpu.sync_copy(shared, allrows_v)
        tot = jnp.zeros((_L,), jnp.float32)
        for ss in range(_B):
            tot = tot + allrows_v[pl.ds(ss * _L, _L)]
        stage_v[...] = jnp.full((_L,), jnp.sum(tot), jnp.float32)
        pltpu.sync_copy(stage_v, out_hbm)


_sc_celoss = functools.partial(
    pl.kernel,
    out_type=jax.ShapeDtypeStruct((_L,), jnp.float32),
    mesh=plsc.VectorSubcoreMesh(
        core_axis_name="c", subcore_axis_name="s", num_cores=1),
    compiler_params=pltpu.CompilerParams(needs_layout_passes=False),
    scratch_types=[
        pltpu.VMEM((2 * _Q,), jnp.float32),
        pltpu.VMEM((_Q,), jnp.int32),
        pltpu.VMEM((_B,), jnp.int32),
        pltpu.VMEM((_L,), jnp.float32),
        pltpu.VMEM((_B * _L,), jnp.float32),
        pltpu.VMEM_SHARED((_B * _L,), jnp.float32),
        pltpu.SemaphoreType.DMA,
        pltpu.SemaphoreType.DMA,
        pltpu.SemaphoreType.DMA,
        pltpu.SemaphoreType.DMA,
        pltpu.SemaphoreType.DMA,
    ],
)(_ce_body)


def kernel(pred, target, object_num):
    pred2 = pred.reshape(_B, 2 * _Q)
    out = _sc_celoss(pred2, target.astype(jnp.int32),
                     object_num.astype(jnp.int32))
    return out[0]


# speedup vs baseline: 1.2127x; 1.0066x over previous
import functools

import jax
import jax.numpy as jnp
from jax import lax
from jax.experimental import pallas as pl
from jax.experimental.pallas import tpu as pltpu
from jax.experimental.pallas import tpu_sc as plsc

_B = 16
_Q = 4096
_L = 16
_NCHUNK = 2
_QC = _Q // _NCHUNK
_NITER = _QC // _L

_LOG1P_COEFS = (
    6.944574454176289e-05,
    0.9962619482337939,
    -0.4664424386275648,
    0.2186654836622196,
    -0.05545931374207994,
)


def _ce_body(pred_hbm, tgt_hbm, objn_hbm, out_hbm,
             row_v, tgt_v, objn_v, stage_v, allrows_v, shared,
             sem_o, sem_p0, sem_t0, sem_p1, sem_t1):
    s = lax.axis_index("s")
    cp_o = pltpu.async_copy(objn_hbm, objn_v, sem_o)
    cp_p0 = pltpu.async_copy(
        pred_hbm.at[s, pl.ds(0, 2 * _QC)], row_v.at[pl.ds(0, 2 * _QC)], sem_p0)
    cp_t0 = pltpu.async_copy(
        tgt_hbm.at[s, pl.ds(0, _QC)], tgt_v.at[pl.ds(0, _QC)], sem_t0)
    cp_p1 = pltpu.async_copy(
        pred_hbm.at[s, pl.ds(2 * _QC, 2 * _QC)],
        row_v.at[pl.ds(2 * _QC, 2 * _QC)], sem_p1)
    cp_t1 = pltpu.async_copy(
        tgt_hbm.at[s, pl.ds(_QC, _QC)], tgt_v.at[pl.ds(_QC, _QC)], sem_t1)

    cp_o.wait()
    iota = lax.broadcasted_iota(jnp.int32, (_L,), 0)
    sful = jnp.full((_L,), s, jnp.int32)
    my_numb = plsc.load_gather(objn_v, [sful])
    inv = (1.0 / _B) / my_numb.astype(jnp.float32)
    two_iota = iota * 2

    def step(i, acc):
        idx0 = two_iota + i * (2 * _L)
        g0 = plsc.load_gather(row_v, [idx0])
        g1 = plsc.load_gather(row_v, [idx0 + 1])
        t = tgt_v[pl.ds(i * _L, _L)]
        m = jnp.maximum(g0, g1)
        e = jnp.exp(jnp.minimum(g0, g1) - m)
        lp = jnp.full((_L,), _LOG1P_COEFS[-1], jnp.float32)
        for q in _LOG1P_COEFS[-2::-1]:
            lp = lp * e + q
        pt = jnp.where(t == 0, g0, g1)
        ce = (m - pt) + lp
        return acc + jnp.where(iota + i * _L < my_numb, ce, 0.0)

    cp_p0.wait()
    cp_t0.wait()
    acc = lax.fori_loop(0, _NITER, step, jnp.zeros((_L,), jnp.float32))
    cp_p1.wait()
    cp_t1.wait()
    acc = lax.fori_loop(_NITER, 2 * _NITER, step, acc)

    stage_v[...] = acc * inv
    pltpu.sync_copy(stage_v, shared.at[pl.ds(s * _L, _L)])
    plsc.subcore_barrier()

    @pl.when(s == 0)
    def _():
        pltpu.sync_copy(shared, allrows_v)
        tot = jnp.zeros((_L,), jnp.float32)
        for ss in range(_B):
            tot = tot + allrows_v[pl.ds(ss * _L, _L)]
        stage_v[...] = jnp.full((_L,), jnp.sum(tot), jnp.float32)
        pltpu.sync_copy(stage_v, out_hbm)


_sc_celoss = functools.partial(
    pl.kernel,
    out_type=jax.ShapeDtypeStruct((_L,), jnp.float32),
    mesh=plsc.VectorSubcoreMesh(
        core_axis_name="c", subcore_axis_name="s", num_cores=1),
    compiler_params=pltpu.CompilerParams(
        needs_layout_passes=False, use_tc_tiling_on_sc=True),
    scratch_types=[
        pltpu.VMEM((2 * _Q,), jnp.float32),
        pltpu.VMEM((_Q,), jnp.int32),
        pltpu.VMEM((_B,), jnp.int32),
        pltpu.VMEM((_L,), jnp.float32),
        pltpu.VMEM((_B * _L,), jnp.float32),
        pltpu.VMEM_SHARED((_B * _L,), jnp.float32),
        pltpu.SemaphoreType.DMA,
        pltpu.SemaphoreType.DMA,
        pltpu.SemaphoreType.DMA,
        pltpu.SemaphoreType.DMA,
        pltpu.SemaphoreType.DMA,
    ],
)(_ce_body)


def kernel(pred, target, object_num):
    pred2 = pred.reshape(_B, 2 * _Q)
    out = _sc_celoss(pred2, target.astype(jnp.int32),
                     object_num.astype(jnp.int32))
    return out[0]


# speedup vs baseline: 1.4773x; 1.2181x over previous
import functools

import jax
import jax.numpy as jnp
from jax import lax
from jax.experimental import pallas as pl
from jax.experimental.pallas import tpu as pltpu
from jax.experimental.pallas import tpu_sc as plsc

_B = 16
_Q = 4096
_L = 16
_NCHUNK = 2
_QC = _Q // _NCHUNK
_NITER = _QC // _L

_LOG1P_COEFS = (
    6.944574454176289e-05,
    0.9962619482337939,
    -0.4664424386275648,
    0.2186654836622196,
    -0.05545931374207994,
)


def _ce_body(pred_hbm, tgt_hbm, objn_hbm, out_hbm,
             row_v, tgt_v, objn_v, stage_v, allrows_v, shared,
             sem_o, sem_p0, sem_t0, sem_p1, sem_t1):
    s = lax.axis_index("s")
    nrow = 2 * _Q // 128
    nrh = nrow // 2
    cp_o = pltpu.async_copy(objn_hbm, objn_v, sem_o)
    cp_p0 = pltpu.async_copy(
        pred_hbm.at[pl.ds(s * nrow, nrh)], row_v.at[pl.ds(0, nrh)], sem_p0)
    cp_t0 = pltpu.async_copy(
        tgt_hbm.at[s, pl.ds(0, _QC)], tgt_v.at[pl.ds(0, _QC)], sem_t0)
    cp_p1 = pltpu.async_copy(
        pred_hbm.at[pl.ds(s * nrow + nrh, nrh)],
        row_v.at[pl.ds(nrh, nrh)], sem_p1)
    cp_t1 = pltpu.async_copy(
        tgt_hbm.at[s, pl.ds(_QC, _QC)], tgt_v.at[pl.ds(_QC, _QC)], sem_t1)

    cp_o.wait()
    iota = lax.broadcasted_iota(jnp.int32, (_L,), 0)
    sful = jnp.full((_L,), s, jnp.int32)
    my_numb = plsc.load_gather(objn_v, [sful])
    inv = (1.0 / _B) / my_numb.astype(jnp.float32)
    two_iota = iota * 2

    def step(i, acc):
        r = (i >> 3) * 2
        j0 = (i & 7) * _L
        g0 = row_v[r, pl.ds(j0, _L)]
        g1 = row_v[r + 1, pl.ds(j0, _L)]
        t = tgt_v[pl.ds(i * _L, _L)]
        m = jnp.maximum(g0, g1)
        e = jnp.exp(jnp.minimum(g0, g1) - m)
        lp = jnp.full((_L,), _LOG1P_COEFS[-1], jnp.float32)
        for q in _LOG1P_COEFS[-2::-1]:
            lp = lp * e + q
        pt = jnp.where(t == 0, g0, g1)
        ce = (m - pt) + lp
        return acc + jnp.where(iota + i * _L < my_numb, ce, 0.0)

    cp_p0.wait()
    cp_t0.wait()
    acc = lax.fori_loop(0, _NITER, step, jnp.zeros((_L,), jnp.float32))
    cp_p1.wait()
    cp_t1.wait()
    acc = lax.fori_loop(_NITER, 2 * _NITER, step, acc)

    stage_v[...] = acc * inv
    pltpu.sync_copy(stage_v, shared.at[pl.ds(s * _L, _L)])
    plsc.subcore_barrier()

    @pl.when(s == 0)
    def _():
        pltpu.sync_copy(shared, allrows_v)
        tot = jnp.zeros((_L,), jnp.float32)
        for ss in range(_B):
            tot = tot + allrows_v[pl.ds(ss * _L, _L)]
        stage_v[...] = jnp.full((_L,), jnp.sum(tot), jnp.float32)
        pltpu.sync_copy(stage_v, out_hbm)


_sc_celoss = functools.partial(
    pl.kernel,
    out_type=jax.ShapeDtypeStruct((_L,), jnp.float32),
    mesh=plsc.VectorSubcoreMesh(
        core_axis_name="c", subcore_axis_name="s", num_cores=1),
    compiler_params=pltpu.CompilerParams(
        needs_layout_passes=False, use_tc_tiling_on_sc=True),
    scratch_types=[
        pltpu.VMEM((2 * _Q // 128, 128), jnp.float32),
        pltpu.VMEM((_Q,), jnp.int32),
        pltpu.VMEM((_B,), jnp.int32),
        pltpu.VMEM((_L,), jnp.float32),
        pltpu.VMEM((_B * _L,), jnp.float32),
        pltpu.VMEM_SHARED((_B * _L,), jnp.float32),
        pltpu.SemaphoreType.DMA,
        pltpu.SemaphoreType.DMA,
        pltpu.SemaphoreType.DMA,
        pltpu.SemaphoreType.DMA,
        pltpu.SemaphoreType.DMA,
    ],
)(_ce_body)


def kernel(pred, target, object_num):
    pred2 = pred.reshape(_B, _Q // 128, 128, 2).transpose(0, 1, 3, 2)
    pred2 = pred2.reshape(_B * (_Q // 128) * 2, 128)
    out = _sc_celoss(pred2, target.astype(jnp.int32),
                     object_num.astype(jnp.int32))
    return out[0]
